# baseline (speedup 1.0000x reference)
import functools

import jax
import jax.numpy as jnp
from jax import lax
from jax.experimental import pallas as pl
from jax.experimental.pallas import tpu as pltpu
from jax.experimental.pallas import tpu_sc as plsc

_VOCAB = 272
_D = 512
_ADDR_KEY = 206
_CODE_START = 256
_CODE_END = 257

_NC = 2
_NS = 16
_NW = _NC * _NS
_CHUNK = 128


def _prep_body(tok_ref, addr_ref):
    tok = tok_ref[...]
    b, s = tok.shape
    pos = lax.broadcasted_iota(jnp.int32, (b, s), 1)
    cs = jnp.where(tok == _CODE_START, pos, -1)
    se = (tok == _CODE_END).astype(jnp.int32)
    k = 1
    while k < s:
        fill_cs = jnp.full((b, k), -1, jnp.int32)
        fill_se = jnp.zeros((b, k), jnp.int32)
        cs = jnp.maximum(cs, jnp.concatenate([fill_cs, cs[:, : s - k]], axis=1))
        se = jnp.maximum(se, jnp.concatenate([fill_se, se[:, : s - k]], axis=1))
        k *= 2
    mask = (tok < 256) & (cs >= 0) & (se == 0)
    seq_pos = jnp.maximum(pos - cs - 1, 0)
    instr = lax.shift_right_logical(seq_pos * 52429, 18)
    byte_off = seq_pos - instr * 5
    addr = instr * 8 + byte_off
    addr_ref[...] = jnp.where(mask, addr, -1)


def _prep(token_ids):
    return pl.pallas_call(
        _prep_body,
        out_shape=jax.ShapeDtypeStruct(token_ids.shape, jnp.int32),
    )(token_ids)


def _sc_body(w_hbm, tok_hbm, addr_hbm, out_hbm, idx_v, addr_v, rows_v, sem):
    wid = lax.axis_index("s") * _NC + lax.axis_index("c")
    n_total = tok_hbm.shape[0]
    n_per_w = n_total // _NW
    base = wid * n_per_w
    nchunks = n_per_w // _CHUNK

    def chunk_step(ci, carry):
        off = base + ci * _CHUNK
        pltpu.sync_copy(tok_hbm.at[pl.ds(off, _CHUNK)], idx_v)
        pltpu.sync_copy(addr_hbm.at[pl.ds(off, _CHUNK)], addr_v)
        pltpu.async_copy(w_hbm.at[idx_v], rows_v, sem).wait()
        ones = jnp.full((16,), 1.0, jnp.float32)
        for g in range(_CHUNK // 16):
            a = addr_v[pl.ds(g * 16, 16)]
            m = a >= 0
            lo = jnp.bitwise_and(a, 15)
            hi = jnp.bitwise_and(lax.shift_right_logical(a, 4), 15)
            top = jnp.bitwise_and(lax.shift_right_logical(a, 8), 15)
            row = lax.broadcasted_iota(jnp.int32, (16,), 0) + (g * 16)
            plsc.store_scatter(rows_v, [row, _ADDR_KEY + lo], ones, mask=m)
            plsc.store_scatter(rows_v, [row, _ADDR_KEY + 16 + hi], ones, mask=m)
            plsc.store_scatter(rows_v, [row, _ADDR_KEY + 32 + top], ones, mask=m)
        pltpu.sync_copy(rows_v, out_hbm.at[pl.ds(off, _CHUNK)])
        return carry

    lax.fori_loop(0, nchunks, chunk_step, 0)


def _sc_gather(w, tok_flat, addr_flat):
    n = tok_flat.shape[0]
    mesh = plsc.VectorSubcoreMesh(
        core_axis_name="c", subcore_axis_name="s",
        num_cores=_NC, num_subcores=_NS,
    )
    return pl.kernel(
        _sc_body,
        out_type=jax.ShapeDtypeStruct((n, _D), jnp.float32),
        mesh=mesh,
        compiler_params=pltpu.CompilerParams(
            use_tc_tiling_on_sc=False, needs_layout_passes=False),
        scratch_types=[
            pltpu.VMEM((_CHUNK,), jnp.int32),
            pltpu.VMEM((_CHUNK,), jnp.int32),
            pltpu.VMEM((_CHUNK, _D), jnp.float32),
            pltpu.SemaphoreType.DMA,
        ],
    )(w, tok_flat, addr_flat)


def kernel(token_ids, W):
    b, s = token_ids.shape
    tok = token_ids.astype(jnp.int32)
    addr = _prep(tok)
    out = _sc_gather(W, tok.reshape(-1), addr.reshape(-1))
    return out.reshape(b, s, _D)


# speedup vs baseline: 1.0163x; 1.0163x over previous
import functools

import jax
import jax.numpy as jnp
from jax import lax
from jax.experimental import pallas as pl
from jax.experimental.pallas import tpu as pltpu
from jax.experimental.pallas import tpu_sc as plsc

_VOCAB = 272
_D = 512
_ADDR_KEY = 206
_CODE_START = 256
_CODE_END = 257

_NC = 2
_NS = 16
_NW = _NC * _NS
_CHUNK = 64
_NBUF = 3


def _prep_body(tok_ref, addr_ref):
    tok = tok_ref[...]
    b, s = tok.shape
    pos = lax.broadcasted_iota(jnp.int32, (b, s), 1)
    cs = jnp.where(tok == _CODE_START, pos, -1)
    se = (tok == _CODE_END).astype(jnp.int32)
    k = 1
    while k < s:
        fill_cs = jnp.full((b, k), -1, jnp.int32)
        fill_se = jnp.zeros((b, k), jnp.int32)
        cs = jnp.maximum(cs, jnp.concatenate([fill_cs, cs[:, : s - k]], axis=1))
        se = jnp.maximum(se, jnp.concatenate([fill_se, se[:, : s - k]], axis=1))
        k *= 2
    mask = (tok < 256) & (cs >= 0) & (se == 0)
    seq_pos = jnp.maximum(pos - cs - 1, 0)
    instr = lax.shift_right_logical(seq_pos * 52429, 18)
    byte_off = seq_pos - instr * 5
    addr = instr * 8 + byte_off
    addr_ref[...] = jnp.where(mask, addr, -1)


def _prep(token_ids):
    return pl.pallas_call(
        _prep_body,
        out_shape=jax.ShapeDtypeStruct(token_ids.shape, jnp.int32),
    )(token_ids)


def _patch_chunk(rows_v, addr_v, ci):
    ones = jnp.full((16,), 1.0, jnp.float32)
    for g in range(_CHUNK // 16):
        a = addr_v[pl.ds(ci * _CHUNK + g * 16, 16)]
        m = a >= 0
        lo = jnp.bitwise_and(a, 15)
        hi = jnp.bitwise_and(lax.shift_right_logical(a, 4), 15)
        top = jnp.bitwise_and(lax.shift_right_logical(a, 8), 15)
        row = lax.broadcasted_iota(jnp.int32, (16,), 0) + (g * 16)
        plsc.store_scatter(rows_v, [row, _ADDR_KEY + lo], ones, mask=m)
        plsc.store_scatter(rows_v, [row, _ADDR_KEY + 16 + hi], ones, mask=m)
        plsc.store_scatter(rows_v, [row, _ADDR_KEY + 32 + top], ones, mask=m)


def _sc_body(w_hbm, tok_hbm, addr_hbm, out_hbm,
             idx_v, addr_v, rows0, rows1, rows2,
             gs0, gs1, gs2, os0, os1, os2):
    wid = lax.axis_index("s") * _NC + lax.axis_index("c")
    n_total = tok_hbm.shape[0]
    n_per_w = n_total // _NW
    base = wid * n_per_w
    nchunks = n_per_w // _CHUNK
    rows = [rows0, rows1, rows2]
    gsem = [gs0, gs1, gs2]
    osem = [os0, os1, os2]

    pltpu.sync_copy(tok_hbm.at[pl.ds(base, n_per_w)], idx_v)
    pltpu.sync_copy(addr_hbm.at[pl.ds(base, n_per_w)], addr_v)

    def fire_gather(ci):
        b = ci % _NBUF
        pltpu.async_copy(
            w_hbm.at[idx_v.at[pl.ds(ci * _CHUNK, _CHUNK)]], rows[b], gsem[b])

    for ci in range(_NBUF):
        fire_gather(ci)
    for ci in range(nchunks):
        b = ci % _NBUF
        pltpu.make_async_copy(
            w_hbm.at[idx_v.at[pl.ds(ci * _CHUNK, _CHUNK)]], rows[b], gsem[b]
        ).wait()
        _patch_chunk(rows[b], addr_v, ci)
        out_slice = out_hbm.at[pl.ds(base + ci * _CHUNK, _CHUNK)]
        pltpu.async_copy(rows[b], out_slice, osem[b])
        if ci + _NBUF < nchunks:
            pltpu.make_async_copy(rows[b], out_slice, osem[b]).wait()
            fire_gather(ci + _NBUF)
    for ci in range(nchunks - _NBUF, nchunks):
        b = ci % _NBUF
        out_slice = out_hbm.at[pl.ds(base + ci * _CHUNK, _CHUNK)]
        pltpu.make_async_copy(rows[b], out_slice, osem[b]).wait()


def _sc_gather(w, tok_flat, addr_flat):
    n = tok_flat.shape[0]
    n_per_w = n // _NW
    mesh = plsc.VectorSubcoreMesh(
        core_axis_name="c", subcore_axis_name="s",
        num_cores=_NC, num_subcores=_NS,
    )
    return pl.kernel(
        _sc_body,
        out_type=jax.ShapeDtypeStruct((n, _D), jnp.float32),
        mesh=mesh,
        compiler_params=pltpu.CompilerParams(
            use_tc_tiling_on_sc=False, needs_layout_passes=False),
        scratch_types=[
            pltpu.VMEM((n_per_w,), jnp.int32),
            pltpu.VMEM((n_per_w,), jnp.int32),
            pltpu.VMEM((_CHUNK, _D), jnp.float32),
            pltpu.VMEM((_CHUNK, _D), jnp.float32),
            pltpu.VMEM((_CHUNK, _D), jnp.float32),
            pltpu.SemaphoreType.DMA,
            pltpu.SemaphoreType.DMA,
            pltpu.SemaphoreType.DMA,
            pltpu.SemaphoreType.DMA,
            pltpu.SemaphoreType.DMA,
            pltpu.SemaphoreType.DMA,
        ],
    )(w, tok_flat, addr_flat)


def kernel(token_ids, W):
    b, s = token_ids.shape
    tok = token_ids.astype(jnp.int32)
    addr = _prep(tok)
    out = _sc_gather(W, tok.reshape(-1), addr.reshape(-1))
    return out.reshape(b, s, _D)


# speedup vs baseline: 1.4541x; 1.4308x over previous
import functools

import jax
import jax.numpy as jnp
from jax import lax
from jax.experimental import pallas as pl
from jax.experimental.pallas import tpu as pltpu
from jax.experimental.pallas import tpu_sc as plsc

_VOCAB = 272
_D = 512
_ADDR_KEY = 206
_CODE_START = 256
_CODE_END = 257

_NC = 2
_NS = 16
_NW = _NC * _NS
_CHUNK = 64
_NBUF = 3


def _prep_body(tok_ref, addr_ref):
    tok = tok_ref[...]
    b, s = tok.shape
    pos = lax.broadcasted_iota(jnp.int32, (b, s), 1)
    cs = jnp.where(tok == _CODE_START, pos, -1)
    se = (tok == _CODE_END).astype(jnp.int32)
    k = 1
    while k < s:
        fill_cs = jnp.full((b, k), -1, jnp.int32)
        fill_se = jnp.zeros((b, k), jnp.int32)
        cs = jnp.maximum(cs, jnp.concatenate([fill_cs, cs[:, : s - k]], axis=1))
        se = jnp.maximum(se, jnp.concatenate([fill_se, se[:, : s - k]], axis=1))
        k *= 2
    mask = (tok < 256) & (cs >= 0) & (se == 0)
    seq_pos = jnp.maximum(pos - cs - 1, 0)
    instr = lax.shift_right_logical(seq_pos * 52429, 18)
    byte_off = seq_pos - instr * 5
    addr = instr * 8 + byte_off
    addr_ref[...] = jnp.where(mask, addr, -1)


def _prep(token_ids):
    return pl.pallas_call(
        _prep_body,
        out_shape=jax.ShapeDtypeStruct(token_ids.shape, jnp.int32),
    )(token_ids)


def _patch_chunk(rows_v, addr_v, ci):
    ones = jnp.full((16,), 1.0, jnp.float32)
    for g in range(_CHUNK // 16):
        a = addr_v[pl.ds(ci * _CHUNK + g * 16, 16)]
        m = a >= 0
        lo = jnp.bitwise_and(a, 15)
        hi = jnp.bitwise_and(lax.shift_right_logical(a, 4), 15)
        top = jnp.bitwise_and(lax.shift_right_logical(a, 8), 15)
        row = lax.broadcasted_iota(jnp.int32, (16,), 0) + (g * 16)
        plsc.store_scatter(rows_v, [row, _ADDR_KEY + lo], ones, mask=m)
        plsc.store_scatter(rows_v, [row, _ADDR_KEY + 16 + hi], ones, mask=m)
        plsc.store_scatter(rows_v, [row, _ADDR_KEY + 32 + top], ones, mask=m)


def _sc_body(w_hbm, tok_hbm, addr_hbm, out_hbm,
             w_sh, idx_v, addr_v, rows0, rows1, rows2,
             gs0, gs1, gs2, os0, os1, os2):
    sid = lax.axis_index("s")
    wid = sid * _NC + lax.axis_index("c")
    n_total = tok_hbm.shape[0]
    n_per_w = n_total // _NW
    base = wid * n_per_w
    nchunks = n_per_w // _CHUNK
    rows = [rows0, rows1, rows2]
    gsem = [gs0, gs1, gs2]
    osem = [os0, os1, os2]

    rows_per_tile = _VOCAB // _NS
    pltpu.sync_copy(w_hbm.at[pl.ds(sid * rows_per_tile, rows_per_tile)],
                    w_sh.at[pl.ds(sid * rows_per_tile, rows_per_tile)])
    pltpu.sync_copy(tok_hbm.at[pl.ds(base, n_per_w)], idx_v)
    pltpu.sync_copy(addr_hbm.at[pl.ds(base, n_per_w)], addr_v)
    plsc.subcore_barrier()

    def fire_gather(ci):
        b = ci % _NBUF
        pltpu.async_copy(
            w_sh.at[idx_v.at[pl.ds(ci * _CHUNK, _CHUNK)]], rows[b], gsem[b])

    for ci in range(_NBUF):
        fire_gather(ci)
    for ci in range(nchunks):
        b = ci % _NBUF
        pltpu.make_async_copy(
            w_sh.at[idx_v.at[pl.ds(ci * _CHUNK, _CHUNK)]], rows[b], gsem[b]
        ).wait()
        _patch_chunk(rows[b], addr_v, ci)
        out_slice = out_hbm.at[pl.ds(base + ci * _CHUNK, _CHUNK)]
        pltpu.async_copy(rows[b], out_slice, osem[b])
        if ci + _NBUF < nchunks:
            pltpu.make_async_copy(rows[b], out_slice, osem[b]).wait()
            fire_gather(ci + _NBUF)
    for ci in range(nchunks - _NBUF, nchunks):
        b = ci % _NBUF
        out_slice = out_hbm.at[pl.ds(base + ci * _CHUNK, _CHUNK)]
        pltpu.make_async_copy(rows[b], out_slice, osem[b]).wait()


def _sc_gather(w, tok_flat, addr_flat):
    n = tok_flat.shape[0]
    n_per_w = n // _NW
    mesh = plsc.VectorSubcoreMesh(
        core_axis_name="c", subcore_axis_name="s",
        num_cores=_NC, num_subcores=_NS,
    )
    return pl.kernel(
        _sc_body,
        out_type=jax.ShapeDtypeStruct((n, _D), jnp.float32),
        mesh=mesh,
        compiler_params=pltpu.CompilerParams(
            use_tc_tiling_on_sc=False, needs_layout_passes=False),
        scratch_types=[
            pltpu.VMEM_SHARED((_VOCAB, _D), jnp.float32),
            pltpu.VMEM((n_per_w,), jnp.int32),
            pltpu.VMEM((n_per_w,), jnp.int32),
            pltpu.VMEM((_CHUNK, _D), jnp.float32),
            pltpu.VMEM((_CHUNK, _D), jnp.float32),
            pltpu.VMEM((_CHUNK, _D), jnp.float32),
            pltpu.SemaphoreType.DMA,
            pltpu.SemaphoreType.DMA,
            pltpu.SemaphoreType.DMA,
            pltpu.SemaphoreType.DMA,
            pltpu.SemaphoreType.DMA,
            pltpu.SemaphoreType.DMA,
        ],
    )(w, tok_flat, addr_flat)


def kernel(token_ids, W):
    b, s = token_ids.shape
    tok = token_ids.astype(jnp.int32)
    addr = _prep(tok)
    out = _sc_gather(W, tok.reshape(-1), addr.reshape(-1))
    return out.reshape(b, s, _D)


# speedup vs baseline: 1.4542x; 1.0001x over previous
import functools

import jax
import jax.numpy as jnp
from jax import lax
from jax.experimental import pallas as pl
from jax.experimental.pallas import tpu as pltpu
from jax.experimental.pallas import tpu_sc as plsc

_VOCAB = 272
_D = 512
_ADDR_KEY = 206
_CODE_START = 256
_CODE_END = 257

_NC = 2
_NS = 16
_NW = _NC * _NS
_CHUNK = 64
_NBUF = 3


def _prep_body(tok_ref, addr_ref):
    tok = tok_ref[...]
    b, s = tok.shape
    pos = lax.broadcasted_iota(jnp.int32, (b, s), 1)
    cs = jnp.where(tok == _CODE_START, pos, -1)
    se = (tok == _CODE_END).astype(jnp.int32)
    k = 1
    while k < s:
        fill_cs = jnp.full((b, k), -1, jnp.int32)
        fill_se = jnp.zeros((b, k), jnp.int32)
        cs = jnp.maximum(cs, jnp.concatenate([fill_cs, cs[:, : s - k]], axis=1))
        se = jnp.maximum(se, jnp.concatenate([fill_se, se[:, : s - k]], axis=1))
        k *= 2
    mask = (tok < 256) & (cs >= 0) & (se == 0)
    seq_pos = jnp.maximum(pos - cs - 1, 0)
    instr = lax.shift_right_logical(seq_pos * 52429, 18)
    byte_off = seq_pos - instr * 5
    addr = instr * 8 + byte_off
    addr_ref[...] = jnp.where(mask, addr, -1)


def _prep(token_ids):
    return pl.pallas_call(
        _prep_body,
        out_shape=jax.ShapeDtypeStruct(token_ids.shape, jnp.int32),
    )(token_ids)


def _patch_chunk(rows_v, addr_v, ci):
    ones = jnp.full((16,), 1.0, jnp.float32)
    for g in range(_CHUNK // 16):
        a = addr_v[pl.ds(ci * _CHUNK + g * 16, 16)]
        m = a >= 0
        lo = jnp.bitwise_and(a, 15)
        hi = jnp.bitwise_and(lax.shift_right_logical(a, 4), 15)
        top = jnp.bitwise_and(lax.shift_right_logical(a, 8), 15)
        row = lax.broadcasted_iota(jnp.int32, (16,), 0) + (g * 16)
        plsc.store_scatter(rows_v, [row, _ADDR_KEY + lo], ones, mask=m)
        plsc.store_scatter(rows_v, [row, _ADDR_KEY + 16 + hi], ones, mask=m)
        plsc.store_scatter(rows_v, [row, _ADDR_KEY + 32 + top], ones, mask=m)


def _sc_body(w_hbm, tok_hbm, addr_hbm, out_hbm,
             w_sh, idx_v, addr_v, rows0, rows1, rows2,
             gs0, gs1, gs2, os0, os1, os2):
    sid = lax.axis_index("s")
    wid = sid * _NC + lax.axis_index("c")
    bsz, seq = tok_hbm.shape
    n_per_w = (bsz * seq) // _NW
    w_per_row = seq // n_per_w
    bi = wid // w_per_row
    s0 = (wid % w_per_row) * n_per_w
    nchunks = n_per_w // _CHUNK
    rows = [rows0, rows1, rows2]
    gsem = [gs0, gs1, gs2]
    osem = [os0, os1, os2]

    rows_per_tile = _VOCAB // _NS
    pltpu.sync_copy(w_hbm.at[pl.ds(sid * rows_per_tile, rows_per_tile)],
                    w_sh.at[pl.ds(sid * rows_per_tile, rows_per_tile)])
    pltpu.sync_copy(tok_hbm.at[bi, pl.ds(s0, n_per_w)], idx_v)
    pltpu.sync_copy(addr_hbm.at[bi, pl.ds(s0, n_per_w)], addr_v)
    plsc.subcore_barrier()

    def fire_gather(ci):
        b = ci % _NBUF
        pltpu.async_copy(
            w_sh.at[idx_v.at[pl.ds(ci * _CHUNK, _CHUNK)]], rows[b], gsem[b])

    for ci in range(_NBUF):
        fire_gather(ci)
    for ci in range(nchunks):
        b = ci % _NBUF
        pltpu.make_async_copy(
            w_sh.at[idx_v.at[pl.ds(ci * _CHUNK, _CHUNK)]], rows[b], gsem[b]
        ).wait()
        _patch_chunk(rows[b], addr_v, ci)
        out_slice = out_hbm.at[bi, pl.ds(s0 + ci * _CHUNK, _CHUNK)]
        pltpu.async_copy(rows[b], out_slice, osem[b])
        if ci + _NBUF < nchunks:
            pltpu.make_async_copy(rows[b], out_slice, osem[b]).wait()
            fire_gather(ci + _NBUF)
    for ci in range(nchunks - _NBUF, nchunks):
        b = ci % _NBUF
        out_slice = out_hbm.at[bi, pl.ds(s0 + ci * _CHUNK, _CHUNK)]
        pltpu.make_async_copy(rows[b], out_slice, osem[b]).wait()


def _sc_gather(w, tok, addr):
    bsz, seq = tok.shape
    n_per_w = (bsz * seq) // _NW
    mesh = plsc.VectorSubcoreMesh(
        core_axis_name="c", subcore_axis_name="s",
        num_cores=_NC, num_subcores=_NS,
    )
    return pl.kernel(
        _sc_body,
        out_type=jax.ShapeDtypeStruct((bsz, seq, _D), jnp.float32),
        mesh=mesh,
        compiler_params=pltpu.CompilerParams(
            use_tc_tiling_on_sc=False, needs_layout_passes=False),
        scratch_types=[
            pltpu.VMEM_SHARED((_VOCAB, _D), jnp.float32),
            pltpu.VMEM((n_per_w,), jnp.int32),
            pltpu.VMEM((n_per_w,), jnp.int32),
            pltpu.VMEM((_CHUNK, _D), jnp.float32),
            pltpu.VMEM((_CHUNK, _D), jnp.float32),
            pltpu.VMEM((_CHUNK, _D), jnp.float32),
            pltpu.SemaphoreType.DMA,
            pltpu.SemaphoreType.DMA,
            pltpu.SemaphoreType.DMA,
            pltpu.SemaphoreType.DMA,
            pltpu.SemaphoreType.DMA,
            pltpu.SemaphoreType.DMA,
        ],
    )(w, tok, addr)


def kernel(token_ids, W):
    tok = token_ids.astype(jnp.int32)
    addr = _prep(tok)
    return _sc_gather(W, tok, addr)


# speedup vs baseline: 1.7361x; 1.1939x over previous
import functools

import jax
import jax.numpy as jnp
from jax import lax
from jax.experimental import pallas as pl
from jax.experimental.pallas import tpu as pltpu
from jax.experimental.pallas import tpu_sc as plsc

_VOCAB = 272
_D = 512
_ADDR_KEY = 206
_CODE_START = 256
_CODE_END = 257

_NC = 2
_NS = 16
_NW = _NC * _NS
_CHUNK = 64
_NBUF = 3


def _prep_body(tok_ref, addr_ref):
    tok = tok_ref[...]
    b, s = tok.shape
    pos = lax.broadcasted_iota(jnp.int32, (b, s), 1)
    cs = jnp.where(tok == _CODE_START, pos, -1)
    se = (tok == _CODE_END).astype(jnp.int32)
    k = 1
    while k < s:
        fill_cs = jnp.full((b, k), -1, jnp.int32)
        fill_se = jnp.zeros((b, k), jnp.int32)
        cs = jnp.maximum(cs, jnp.concatenate([fill_cs, cs[:, : s - k]], axis=1))
        se = jnp.maximum(se, jnp.concatenate([fill_se, se[:, : s - k]], axis=1))
        k *= 2
    mask = (tok < 256) & (cs >= 0) & (se == 0)
    seq_pos = jnp.maximum(pos - cs - 1, 0)
    instr = lax.shift_right_logical(seq_pos * 52429, 18)
    byte_off = seq_pos - instr * 5
    addr = instr * 8 + byte_off
    addr_ref[...] = jnp.where(mask, addr, -1)


def _prep(token_ids):
    return pl.pallas_call(
        _prep_body,
        out_shape=jax.ShapeDtypeStruct(token_ids.shape, jnp.int32),
    )(token_ids)


def _patch_chunk(rows_v, addr_v, ci):
    ones = jnp.full((16,), 1.0, jnp.float32)
    for g in range(_CHUNK // 16):
        a = addr_v[pl.ds(ci * _CHUNK + g * 16, 16)]
        m = a >= 0
        lo = jnp.bitwise_and(a, 15)
        hi = jnp.bitwise_and(lax.shift_right_logical(a, 4), 15)
        top = jnp.bitwise_and(lax.shift_right_logical(a, 8), 15)
        row = lax.broadcasted_iota(jnp.int32, (16,), 0) + (g * 16)
        plsc.store_scatter(rows_v, [row, _ADDR_KEY + lo], ones, mask=m)
        plsc.store_scatter(rows_v, [row, _ADDR_KEY + 16 + hi], ones, mask=m)
        plsc.store_scatter(rows_v, [row, _ADDR_KEY + 32 + top], ones, mask=m)


def _sc_body(seq, w_hbm, tok_hbm, addr_hbm, out_hbm,
             idx_v, addr_v, rows0, rows1, rows2,
             gs0, gs1, gs2, os0, os1, os2):
    sid = lax.axis_index("s")
    wid = sid * _NC + lax.axis_index("c")
    n_total = tok_hbm.shape[0]
    n_per_w = n_total // _NW
    w_per_row = seq // n_per_w
    bi = wid // w_per_row
    s0 = (wid % w_per_row) * n_per_w
    base = wid * n_per_w
    nchunks = n_per_w // _CHUNK
    rows = [rows0, rows1, rows2]
    gsem = [gs0, gs1, gs2]
    osem = [os0, os1, os2]

    pltpu.sync_copy(tok_hbm.at[pl.ds(base, n_per_w)], idx_v)
    pltpu.sync_copy(addr_hbm.at[pl.ds(base, n_per_w)], addr_v)

    def fire_gather(ci):
        b = ci % _NBUF
        pltpu.async_copy(
            w_hbm.at[idx_v.at[pl.ds(ci * _CHUNK, _CHUNK)]], rows[b], gsem[b])

    for ci in range(_NBUF):
        fire_gather(ci)
    for ci in range(nchunks):
        b = ci % _NBUF
        pltpu.make_async_copy(
            w_hbm.at[idx_v.at[pl.ds(ci * _CHUNK, _CHUNK)]], rows[b], gsem[b]
        ).wait()
        _patch_chunk(rows[b], addr_v, ci)
        out_slice = out_hbm.at[bi, pl.ds(s0 + ci * _CHUNK, _CHUNK)]
        pltpu.async_copy(rows[b], out_slice, osem[b])
        if ci + _NBUF < nchunks:
            pltpu.make_async_copy(rows[b], out_slice, osem[b]).wait()
            fire_gather(ci + _NBUF)
    for ci in range(nchunks - _NBUF, nchunks):
        b = ci % _NBUF
        out_slice = out_hbm.at[bi, pl.ds(s0 + ci * _CHUNK, _CHUNK)]
        pltpu.make_async_copy(rows[b], out_slice, osem[b]).wait()


def _sc_gather(w, tok, addr, bsz, seq):
    n_per_w = (bsz * seq) // _NW
    mesh = plsc.VectorSubcoreMesh(
        core_axis_name="c", subcore_axis_name="s",
        num_cores=_NC, num_subcores=_NS,
    )
    return pl.kernel(
        functools.partial(_sc_body, seq),
        out_type=jax.ShapeDtypeStruct((bsz, seq, _D), jnp.float32),
        mesh=mesh,
        compiler_params=pltpu.CompilerParams(
            use_tc_tiling_on_sc=True, needs_layout_passes=False),
        scratch_types=[
            pltpu.VMEM((n_per_w,), jnp.int32),
            pltpu.VMEM((n_per_w,), jnp.int32),
            pltpu.VMEM((_CHUNK, _D), jnp.float32),
            pltpu.VMEM((_CHUNK, _D), jnp.float32),
            pltpu.VMEM((_CHUNK, _D), jnp.float32),
            pltpu.SemaphoreType.DMA,
            pltpu.SemaphoreType.DMA,
            pltpu.SemaphoreType.DMA,
            pltpu.SemaphoreType.DMA,
            pltpu.SemaphoreType.DMA,
            pltpu.SemaphoreType.DMA,
        ],
    )(w, tok, addr)


def kernel(token_ids, W):
    bsz, seq = token_ids.shape
    tok = token_ids.astype(jnp.int32)
    addr = _prep(tok)
    return _sc_gather(W, tok.reshape(-1), addr.reshape(-1), bsz, seq)


# speedup vs baseline: 2.2817x; 1.3142x over previous
import functools

import jax
import jax.numpy as jnp
from jax import lax
from jax.experimental import pallas as pl
from jax.experimental.pallas import tpu as pltpu
from jax.experimental.pallas import tpu_sc as plsc

_VOCAB = 272
_D = 512
_ADDR_KEY = 206
_CODE_START = 256
_CODE_END = 257

_NC = 2
_NS = 16
_NW = _NC * _NS
_CHUNK = 64
_NBUF = 3
_NREP = 32


def _prep_body(tok_ref, w_ref, addr_ref, idxrep_ref, wrep_ref):
    tok = tok_ref[...]
    b, s = tok.shape
    pos = lax.broadcasted_iota(jnp.int32, (b, s), 1)
    cs = jnp.where(tok == _CODE_START, pos, -1)
    se = (tok == _CODE_END).astype(jnp.int32)
    k = 1
    while k < s:
        fill_cs = jnp.full((b, k), -1, jnp.int32)
        fill_se = jnp.zeros((b, k), jnp.int32)
        cs = jnp.maximum(cs, jnp.concatenate([fill_cs, cs[:, : s - k]], axis=1))
        se = jnp.maximum(se, jnp.concatenate([fill_se, se[:, : s - k]], axis=1))
        k *= 2
    mask = (tok < 256) & (cs >= 0) & (se == 0)
    seq_pos = jnp.maximum(pos - cs - 1, 0)
    instr = lax.shift_right_logical(seq_pos * 52429, 18)
    byte_off = seq_pos - instr * 5
    addr = instr * 8 + byte_off
    addr_ref[...] = jnp.where(mask, addr, -1)
    n_per_w = (b * s) // _NW
    bi = lax.broadcasted_iota(jnp.int32, (b, s), 0)
    worker = (bi * s + pos) // n_per_w
    idxrep_ref[...] = tok + (worker % _NREP) * _VOCAB
    wrep_ref[...] = jnp.broadcast_to(w_ref[...][None], (_NREP,) + w_ref.shape)


def _prep(token_ids, w):
    return pl.pallas_call(
        _prep_body,
        out_shape=(
            jax.ShapeDtypeStruct(token_ids.shape, jnp.int32),
            jax.ShapeDtypeStruct(token_ids.shape, jnp.int32),
            jax.ShapeDtypeStruct((_NREP,) + w.shape, jnp.float32),
        ),
    )(token_ids, w)


def _patch_chunk(rows_v, addr_v, ci):
    ones = jnp.full((16,), 1.0, jnp.float32)
    for g in range(_CHUNK // 16):
        a = addr_v[pl.ds(ci * _CHUNK + g * 16, 16)]
        m = a >= 0
        lo = jnp.bitwise_and(a, 15)
        hi = jnp.bitwise_and(lax.shift_right_logical(a, 4), 15)
        top = jnp.bitwise_and(lax.shift_right_logical(a, 8), 15)
        row = lax.broadcasted_iota(jnp.int32, (16,), 0) + (g * 16)
        plsc.store_scatter(rows_v, [row, _ADDR_KEY + lo], ones, mask=m)
        plsc.store_scatter(rows_v, [row, _ADDR_KEY + 16 + hi], ones, mask=m)
        plsc.store_scatter(rows_v, [row, _ADDR_KEY + 32 + top], ones, mask=m)


def _sc_body(seq, w_hbm, tok_hbm, addr_hbm, out_hbm,
             idx_v, addr_v, rows0, rows1, rows2,
             gs0, gs1, gs2, os0, os1, os2):
    sid = lax.axis_index("s")
    wid = sid * _NC + lax.axis_index("c")
    n_total = tok_hbm.shape[0]
    n_per_w = n_total // _NW
    w_per_row = seq // n_per_w
    bi = wid // w_per_row
    s0 = (wid % w_per_row) * n_per_w
    base = wid * n_per_w
    nchunks = n_per_w // _CHUNK
    rows = [rows0, rows1, rows2]
    gsem = [gs0, gs1, gs2]
    osem = [os0, os1, os2]

    pltpu.sync_copy(tok_hbm.at[pl.ds(base, n_per_w)], idx_v)
    pltpu.sync_copy(addr_hbm.at[pl.ds(base, n_per_w)], addr_v)

    def fire_gather(ci):
        b = ci % _NBUF
        pltpu.async_copy(
            w_hbm.at[idx_v.at[pl.ds(ci * _CHUNK, _CHUNK)]], rows[b], gsem[b])

    for ci in range(_NBUF):
        fire_gather(ci)
    for ci in range(nchunks):
        b = ci % _NBUF
        pltpu.make_async_copy(
            w_hbm.at[idx_v.at[pl.ds(ci * _CHUNK, _CHUNK)]], rows[b], gsem[b]
        ).wait()
        _patch_chunk(rows[b], addr_v, ci)
        out_slice = out_hbm.at[bi, pl.ds(s0 + ci * _CHUNK, _CHUNK)]
        pltpu.async_copy(rows[b], out_slice, osem[b])
        if ci + _NBUF < nchunks:
            pltpu.make_async_copy(rows[b], out_slice, osem[b]).wait()
            fire_gather(ci + _NBUF)
    for ci in range(nchunks - _NBUF, nchunks):
        b = ci % _NBUF
        out_slice = out_hbm.at[bi, pl.ds(s0 + ci * _CHUNK, _CHUNK)]
        pltpu.make_async_copy(rows[b], out_slice, osem[b]).wait()


def _sc_gather(w, tok, addr, bsz, seq):
    n_per_w = (bsz * seq) // _NW
    mesh = plsc.VectorSubcoreMesh(
        core_axis_name="c", subcore_axis_name="s",
        num_cores=_NC, num_subcores=_NS,
    )
    return pl.kernel(
        functools.partial(_sc_body, seq),
        out_type=jax.ShapeDtypeStruct((bsz, seq, _D), jnp.float32),
        mesh=mesh,
        compiler_params=pltpu.CompilerParams(
            use_tc_tiling_on_sc=True, needs_layout_passes=False),
        scratch_types=[
            pltpu.VMEM((n_per_w,), jnp.int32),
            pltpu.VMEM((n_per_w,), jnp.int32),
            pltpu.VMEM((_CHUNK, _D), jnp.float32),
            pltpu.VMEM((_CHUNK, _D), jnp.float32),
            pltpu.VMEM((_CHUNK, _D), jnp.float32),
            pltpu.SemaphoreType.DMA,
            pltpu.SemaphoreType.DMA,
            pltpu.SemaphoreType.DMA,
            pltpu.SemaphoreType.DMA,
            pltpu.SemaphoreType.DMA,
            pltpu.SemaphoreType.DMA,
        ],
    )(w, tok, addr)


def kernel(token_ids, W):
    bsz, seq = token_ids.shape
    tok = token_ids.astype(jnp.int32)
    addr, idxrep, wrep = _prep(tok, W)
    return _sc_gather(wrep.reshape(_NREP * _VOCAB, _D),
                      idxrep.reshape(-1), addr.reshape(-1), bsz, seq)


# speedup vs baseline: 2.3275x; 1.0201x over previous
import functools

import jax
import jax.numpy as jnp
from jax import lax
from jax.experimental import pallas as pl
from jax.experimental.pallas import tpu as pltpu
from jax.experimental.pallas import tpu_sc as plsc

_VOCAB = 272
_D = 512
_ADDR_KEY = 206
_CODE_START = 256
_CODE_END = 257

_NC = 2
_NS = 16
_NW = _NC * _NS
_CHUNK = 64
_NBUF = 3
_NREP = 16


def _prep_body(tok_ref, w_ref, addr_ref, idxrep_ref, wrep_ref):
    tok = tok_ref[...]
    b, s = tok.shape
    pos = lax.broadcasted_iota(jnp.int32, (b, s), 1)
    cs = jnp.where(tok == _CODE_START, pos, -1)
    se = (tok == _CODE_END).astype(jnp.int32)
    k = 1
    while k < s:
        fill_cs = jnp.full((b, k), -1, jnp.int32)
        fill_se = jnp.zeros((b, k), jnp.int32)
        cs = jnp.maximum(cs, jnp.concatenate([fill_cs, cs[:, : s - k]], axis=1))
        se = jnp.maximum(se, jnp.concatenate([fill_se, se[:, : s - k]], axis=1))
        k *= 2
    mask = (tok < 256) & (cs >= 0) & (se == 0)
    seq_pos = jnp.maximum(pos - cs - 1, 0)
    instr = lax.shift_right_logical(seq_pos * 52429, 18)
    byte_off = seq_pos - instr * 5
    addr = instr * 8 + byte_off
    addr_ref[...] = jnp.where(mask, addr, -1)
    n_per_w = (b * s) // _NW
    bi = lax.broadcasted_iota(jnp.int32, (b, s), 0)
    worker = (bi * s + pos) // n_per_w
    idxrep_ref[...] = tok + (worker % _NREP) * _VOCAB
    wrep_ref[...] = jnp.broadcast_to(w_ref[...][None], (_NREP,) + w_ref.shape)


def _prep(token_ids, w):
    return pl.pallas_call(
        _prep_body,
        out_shape=(
            jax.ShapeDtypeStruct(token_ids.shape, jnp.int32),
            jax.ShapeDtypeStruct(token_ids.shape, jnp.int32),
            jax.ShapeDtypeStruct((_NREP,) + w.shape, jnp.float32),
        ),
    )(token_ids, w)


def _patch_chunk(rows_v, addr_v, ci):
    ones = jnp.full((16,), 1.0, jnp.float32)
    for g in range(_CHUNK // 16):
        a = addr_v[pl.ds(ci * _CHUNK + g * 16, 16)]
        m = a >= 0
        lo = jnp.bitwise_and(a, 15)
        hi = jnp.bitwise_and(lax.shift_right_logical(a, 4), 15)
        top = jnp.bitwise_and(lax.shift_right_logical(a, 8), 15)
        row = lax.broadcasted_iota(jnp.int32, (16,), 0) + (g * 16)
        plsc.store_scatter(rows_v, [row, _ADDR_KEY + lo], ones, mask=m)
        plsc.store_scatter(rows_v, [row, _ADDR_KEY + 16 + hi], ones, mask=m)
        plsc.store_scatter(rows_v, [row, _ADDR_KEY + 32 + top], ones, mask=m)


def _sc_body(seq, w_hbm, tok_hbm, addr_hbm, out_hbm,
             idx_v, addr_v, rows0, rows1, rows2,
             gs0, gs1, gs2, os0, os1, os2):
    sid = lax.axis_index("s")
    wid = sid * _NC + lax.axis_index("c")
    n_total = tok_hbm.shape[0]
    n_per_w = n_total // _NW
    w_per_row = seq // n_per_w
    bi = wid // w_per_row
    s0 = (wid % w_per_row) * n_per_w
    base = wid * n_per_w
    nchunks = n_per_w // _CHUNK
    rows = [rows0, rows1, rows2]
    gsem = [gs0, gs1, gs2]
    osem = [os0, os1, os2]

    pltpu.sync_copy(tok_hbm.at[pl.ds(base, n_per_w)], idx_v)
    pltpu.sync_copy(addr_hbm.at[pl.ds(base, n_per_w)], addr_v)

    def fire_gather(ci):
        b = ci % _NBUF
        pltpu.async_copy(
            w_hbm.at[idx_v.at[pl.ds(ci * _CHUNK, _CHUNK)]], rows[b], gsem[b])

    def out_slice(ci):
        return out_hbm.at[bi, pl.ds(s0 + ci * _CHUNK, _CHUNK)]

    def wait_out(ci):
        b = ci % _NBUF
        pltpu.make_async_copy(rows[b], out_slice(ci), osem[b]).wait()

    for ci in range(_NBUF):
        fire_gather(ci)
    for ci in range(nchunks):
        b = ci % _NBUF
        if ci >= _NBUF - 1 and ci + 1 < nchunks:
            wait_out(ci + 1 - _NBUF)
            fire_gather(ci + 1)
        pltpu.make_async_copy(
            w_hbm.at[idx_v.at[pl.ds(ci * _CHUNK, _CHUNK)]], rows[b], gsem[b]
        ).wait()
        _patch_chunk(rows[b], addr_v, ci)
        pltpu.async_copy(rows[b], out_slice(ci), osem[b])
    for ci in range(nchunks - _NBUF, nchunks):
        wait_out(ci)


def _sc_gather(w, tok, addr, bsz, seq):
    n_per_w = (bsz * seq) // _NW
    mesh = plsc.VectorSubcoreMesh(
        core_axis_name="c", subcore_axis_name="s",
        num_cores=_NC, num_subcores=_NS,
    )
    return pl.kernel(
        functools.partial(_sc_body, seq),
        out_type=jax.ShapeDtypeStruct((bsz, seq, _D), jnp.float32),
        mesh=mesh,
        compiler_params=pltpu.CompilerParams(
            use_tc_tiling_on_sc=True, needs_layout_passes=False),
        scratch_types=[
            pltpu.VMEM((n_per_w,), jnp.int32),
            pltpu.VMEM((n_per_w,), jnp.int32),
            pltpu.VMEM((_CHUNK, _D), jnp.float32),
            pltpu.VMEM((_CHUNK, _D), jnp.float32),
            pltpu.VMEM((_CHUNK, _D), jnp.float32),
            pltpu.SemaphoreType.DMA,
            pltpu.SemaphoreType.DMA,
            pltpu.SemaphoreType.DMA,
            pltpu.SemaphoreType.DMA,
            pltpu.SemaphoreType.DMA,
            pltpu.SemaphoreType.DMA,
        ],
    )(w, tok, addr)


def kernel(token_ids, W):
    bsz, seq = token_ids.shape
    tok = token_ids.astype(jnp.int32)
    addr, idxrep, wrep = _prep(tok, W)
    return _sc_gather(wrep.reshape(_NREP * _VOCAB, _D),
                      idxrep.reshape(-1), addr.reshape(-1), bsz, seq)


# speedup vs baseline: 2.9110x; 1.2507x over previous
import functools

import jax
import jax.numpy as jnp
from jax import lax
from jax.experimental import pallas as pl
from jax.experimental.pallas import tpu as pltpu
from jax.experimental.pallas import tpu_sc as plsc

_VOCAB = 272
_D = 512
_ADDR_KEY = 206
_CODE_START = 256
_CODE_END = 257

_NC = 2
_NS = 16
_NW = _NC * _NS
_CHUNK = 64
_NBUF = 3
_NREP = 16


def _prep_body(tok_ref, addr_ref):
    tok = tok_ref[...]
    b, s = tok.shape
    pos = lax.broadcasted_iota(jnp.int32, (b, s), 1)
    cs = jnp.where(tok == _CODE_START, pos, -1)
    se = (tok == _CODE_END).astype(jnp.int32)
    k = 1
    while k < s:
        fill_cs = jnp.full((b, k), -1, jnp.int32)
        fill_se = jnp.zeros((b, k), jnp.int32)
        cs = jnp.maximum(cs, jnp.concatenate([fill_cs, cs[:, : s - k]], axis=1))
        se = jnp.maximum(se, jnp.concatenate([fill_se, se[:, : s - k]], axis=1))
        k *= 2
    mask = (tok < 256) & (cs >= 0) & (se == 0)
    seq_pos = jnp.maximum(pos - cs - 1, 0)
    instr = lax.shift_right_logical(seq_pos * 52429, 18)
    byte_off = seq_pos - instr * 5
    addr = instr * 8 + byte_off
    addr_ref[...] = jnp.where(mask, addr, -1)


def _prep(token_ids):
    return pl.pallas_call(
        _prep_body,
        out_shape=jax.ShapeDtypeStruct(token_ids.shape, jnp.int32),
    )(token_ids)


def _patch_chunk(rows_v, addr_v, ci):
    ones = jnp.full((16,), 1.0, jnp.float32)
    for g in range(_CHUNK // 16):
        a = addr_v[pl.ds(ci * _CHUNK + g * 16, 16)]
        m = a >= 0
        lo = jnp.bitwise_and(a, 15)
        hi = jnp.bitwise_and(lax.shift_right_logical(a, 4), 15)
        top = jnp.bitwise_and(lax.shift_right_logical(a, 8), 15)
        row = lax.broadcasted_iota(jnp.int32, (16,), 0) + (g * 16)
        plsc.store_scatter(rows_v, [row, _ADDR_KEY + lo], ones, mask=m)
        plsc.store_scatter(rows_v, [row, _ADDR_KEY + 16 + hi], ones, mask=m)
        plsc.store_scatter(rows_v, [row, _ADDR_KEY + 32 + top], ones, mask=m)


def _sc_body(seq, w_hbm, tok_hbm, addr_hbm, out_hbm,
             w_sh, idx_v, addr_v, rows0, rows1, rows2,
             gs0, gs1, gs2, os0, os1, os2):
    sid = lax.axis_index("s")
    wid = sid * _NC + lax.axis_index("c")
    n_total = tok_hbm.shape[0]
    n_per_w = n_total // _NW
    w_per_row = seq // n_per_w
    bi = wid // w_per_row
    s0 = (wid % w_per_row) * n_per_w
    base = wid * n_per_w
    nchunks = n_per_w // _CHUNK
    rows = [rows0, rows1, rows2]
    gsem = [gs0, gs1, gs2]
    osem = [os0, os1, os2]

    pltpu.sync_copy(w_hbm.at[pl.ds(sid * 16, 16)], w_sh.at[pl.ds(sid * 16, 16)])

    @pl.when(sid == 0)
    def _tail():
        pltpu.sync_copy(w_hbm.at[pl.ds(_VOCAB - 16, 16)],
                        w_sh.at[pl.ds(_VOCAB - 16, 16)])

    pltpu.sync_copy(tok_hbm.at[pl.ds(base, n_per_w)], idx_v)
    pltpu.sync_copy(addr_hbm.at[pl.ds(base, n_per_w)], addr_v)
    plsc.subcore_barrier()

    def fire_gather(ci):
        b = ci % _NBUF
        base_t = ci * _CHUNK

        def group(g, carry):
            off = base_t + g * 16
            v = idx_v[pl.ds(off, 16)]
            tl = g * 16
            for k in range(16):
                pltpu.async_copy(w_sh.at[v[k]], rows[b].at[tl + k], gsem[b])
            return carry

        lax.fori_loop(0, _CHUNK // 16, group, 0)

    def wait_gather(ci):
        b = ci % _NBUF
        pltpu.make_async_copy(w_hbm.at[pl.ds(0, _CHUNK)], rows[b],
                              gsem[b]).wait()

    def out_slice(ci):
        return out_hbm.at[bi, pl.ds(s0 + ci * _CHUNK, _CHUNK)]

    def wait_out(ci):
        b = ci % _NBUF
        pltpu.make_async_copy(rows[b], out_slice(ci), osem[b]).wait()

    for ci in range(_NBUF):
        fire_gather(ci)
    for ci in range(nchunks):
        b = ci % _NBUF
        if ci >= _NBUF - 1 and ci + 1 < nchunks:
            wait_out(ci + 1 - _NBUF)
            fire_gather(ci + 1)
        wait_gather(ci)
        _patch_chunk(rows[b], addr_v, ci)
        pltpu.async_copy(rows[b], out_slice(ci), osem[b])
    for ci in range(nchunks - _NBUF, nchunks):
        wait_out(ci)


def _sc_gather(w, tok, addr, bsz, seq):
    n_per_w = (bsz * seq) // _NW
    mesh = plsc.VectorSubcoreMesh(
        core_axis_name="c", subcore_axis_name="s",
        num_cores=_NC, num_subcores=_NS,
    )
    return pl.kernel(
        functools.partial(_sc_body, seq),
        out_type=jax.ShapeDtypeStruct((bsz, seq, _D), jnp.float32),
        mesh=mesh,
        compiler_params=pltpu.CompilerParams(
            use_tc_tiling_on_sc=True, needs_layout_passes=False),
        scratch_types=[
            pltpu.VMEM_SHARED((_VOCAB, _D), jnp.float32),
            pltpu.VMEM((n_per_w,), jnp.int32),
            pltpu.VMEM((n_per_w,), jnp.int32),
            pltpu.VMEM((_CHUNK, _D), jnp.float32),
            pltpu.VMEM((_CHUNK, _D), jnp.float32),
            pltpu.VMEM((_CHUNK, _D), jnp.float32),
            pltpu.SemaphoreType.DMA,
            pltpu.SemaphoreType.DMA,
            pltpu.SemaphoreType.DMA,
            pltpu.SemaphoreType.DMA,
            pltpu.SemaphoreType.DMA,
            pltpu.SemaphoreType.DMA,
        ],
    )(w, tok, addr)


def kernel(token_ids, W):
    bsz, seq = token_ids.shape
    tok = token_ids.astype(jnp.int32)
    addr = _prep(tok)
    return _sc_gather(W, tok.reshape(-1), addr.reshape(-1), bsz, seq)


# speedup vs baseline: 2.9112x; 1.0000x over previous
import functools

import jax
import jax.numpy as jnp
from jax import lax
from jax.experimental import pallas as pl
from jax.experimental.pallas import tpu as pltpu
from jax.experimental.pallas import tpu_sc as plsc

_VOCAB = 272
_D = 512
_ADDR_KEY = 206
_CODE_START = 256
_CODE_END = 257

_NC = 2
_NS = 16
_NW = _NC * _NS
_CHUNK = 64
_NBUF = 3
_NREP = 16


def _prep_body(tok_ref, addr_ref):
    tok = tok_ref[...]
    b, s = tok.shape
    pos = lax.broadcasted_iota(jnp.int32, (b, s), 1)
    cs = jnp.where(tok == _CODE_START, pos, -1)
    se = (tok == _CODE_END).astype(jnp.int32)
    k = 1
    while k < s:
        fill_cs = jnp.full((b, k), -1, jnp.int32)
        fill_se = jnp.zeros((b, k), jnp.int32)
        cs = jnp.maximum(cs, jnp.concatenate([fill_cs, cs[:, : s - k]], axis=1))
        se = jnp.maximum(se, jnp.concatenate([fill_se, se[:, : s - k]], axis=1))
        k *= 2
    mask = (tok < 256) & (cs >= 0) & (se == 0)
    seq_pos = jnp.maximum(pos - cs - 1, 0)
    instr = lax.shift_right_logical(seq_pos * 52429, 18)
    byte_off = seq_pos - instr * 5
    addr = instr * 8 + byte_off
    addr_ref[...] = jnp.where(mask, addr, -1)


def _prep(token_ids):
    return pl.pallas_call(
        _prep_body,
        out_shape=jax.ShapeDtypeStruct(token_ids.shape, jnp.int32),
    )(token_ids)


def _patch_chunk(rows_v, addr_v, ci):
    ones = jnp.full((16,), 1.0, jnp.float32)
    for g in range(_CHUNK // 16):
        a = addr_v[pl.ds(ci * _CHUNK + g * 16, 16)]
        m = a >= 0
        lo = jnp.bitwise_and(a, 15)
        hi = jnp.bitwise_and(lax.shift_right_logical(a, 4), 15)
        top = jnp.bitwise_and(lax.shift_right_logical(a, 8), 15)
        row = lax.broadcasted_iota(jnp.int32, (16,), 0) + (g * 16)
        plsc.store_scatter(rows_v, [row, _ADDR_KEY + lo], ones, mask=m)
        plsc.store_scatter(rows_v, [row, _ADDR_KEY + 16 + hi], ones, mask=m)
        plsc.store_scatter(rows_v, [row, _ADDR_KEY + 32 + top], ones, mask=m)


def _sc_body(seq, w_hbm, tok_hbm, addr_hbm, out_hbm,
             w_sh, idx_v, addr_v, rows0, rows1, rows2,
             gs0, gs1, gs2, os0, os1, os2):
    sid = lax.axis_index("s")
    wid = sid * _NC + lax.axis_index("c")
    bsz, seq_ = tok_hbm.shape
    n_per_w = (bsz * seq_) // _NW
    w_per_row = seq_ // n_per_w
    bi = wid // w_per_row
    s0 = (wid % w_per_row) * n_per_w
    nchunks = n_per_w // _CHUNK
    rows = [rows0, rows1, rows2]
    gsem = [gs0, gs1, gs2]
    osem = [os0, os1, os2]

    pltpu.sync_copy(w_hbm.at[pl.ds(sid * 16, 16)], w_sh.at[pl.ds(sid * 16, 16)])

    @pl.when(sid == 0)
    def _tail():
        pltpu.sync_copy(w_hbm.at[pl.ds(_VOCAB - 16, 16)],
                        w_sh.at[pl.ds(_VOCAB - 16, 16)])

    pltpu.sync_copy(tok_hbm.at[bi, pl.ds(s0, n_per_w)], idx_v)
    pltpu.sync_copy(addr_hbm.at[bi, pl.ds(s0, n_per_w)], addr_v)
    plsc.subcore_barrier()

    def fire_gather(ci):
        b = ci % _NBUF
        base_t = ci * _CHUNK

        def group(g, carry):
            off = base_t + g * 16
            v = idx_v[pl.ds(off, 16)]
            tl = g * 16
            for k in range(16):
                pltpu.async_copy(w_sh.at[v[k]], rows[b].at[tl + k], gsem[b])
            return carry

        lax.fori_loop(0, _CHUNK // 16, group, 0)

    def wait_gather(ci):
        b = ci % _NBUF
        pltpu.make_async_copy(w_hbm.at[pl.ds(0, _CHUNK)], rows[b],
                              gsem[b]).wait()

    def out_slice(ci):
        return out_hbm.at[bi, pl.ds(s0 + ci * _CHUNK, _CHUNK)]

    def wait_out(ci):
        b = ci % _NBUF
        pltpu.make_async_copy(rows[b], out_slice(ci), osem[b]).wait()

    for ci in range(_NBUF):
        fire_gather(ci)
    for ci in range(nchunks):
        b = ci % _NBUF
        if ci >= _NBUF - 1 and ci + 1 < nchunks:
            wait_out(ci + 1 - _NBUF)
            fire_gather(ci + 1)
        wait_gather(ci)
        _patch_chunk(rows[b], addr_v, ci)
        pltpu.async_copy(rows[b], out_slice(ci), osem[b])
    for ci in range(nchunks - _NBUF, nchunks):
        wait_out(ci)


def _sc_gather(w, tok, addr, bsz, seq):
    n_per_w = (bsz * seq) // _NW
    mesh = plsc.VectorSubcoreMesh(
        core_axis_name="c", subcore_axis_name="s",
        num_cores=_NC, num_subcores=_NS,
    )
    return pl.kernel(
        functools.partial(_sc_body, seq),
        out_type=jax.ShapeDtypeStruct((bsz, seq, _D), jnp.float32),
        mesh=mesh,
        compiler_params=pltpu.CompilerParams(
            use_tc_tiling_on_sc=True, needs_layout_passes=False),
        scratch_types=[
            pltpu.VMEM_SHARED((_VOCAB, _D), jnp.float32),
            pltpu.VMEM((n_per_w,), jnp.int32),
            pltpu.VMEM((n_per_w,), jnp.int32),
            pltpu.VMEM((_CHUNK, _D), jnp.float32),
            pltpu.VMEM((_CHUNK, _D), jnp.float32),
            pltpu.VMEM((_CHUNK, _D), jnp.float32),
            pltpu.SemaphoreType.DMA,
            pltpu.SemaphoreType.DMA,
            pltpu.SemaphoreType.DMA,
            pltpu.SemaphoreType.DMA,
            pltpu.SemaphoreType.DMA,
            pltpu.SemaphoreType.DMA,
        ],
    )(w, tok, addr)


def kernel(token_ids, W):
    bsz, seq = token_ids.shape
    tok = token_ids.astype(jnp.int32)
    addr = _prep(tok)
    return _sc_gather(W, tok, addr, bsz, seq)


# speedup vs baseline: 2.9185x; 1.0025x over previous
import functools

import jax
import jax.numpy as jnp
from jax import lax
from jax.experimental import pallas as pl
from jax.experimental.pallas import tpu as pltpu
from jax.experimental.pallas import tpu_sc as plsc

_VOCAB = 272
_D = 512
_ADDR_KEY = 206
_CODE_START = 256
_CODE_END = 257

_NC = 2
_NS = 16
_NW = _NC * _NS
_CHUNK = 32
_NBUF = 6
_NREP = 16


def _prep_body(tok_ref, addr_ref):
    tok = tok_ref[...]
    b, s = tok.shape
    pos = lax.broadcasted_iota(jnp.int32, (b, s), 1)
    cs = jnp.where(tok == _CODE_START, pos, -1)
    se = (tok == _CODE_END).astype(jnp.int32)
    k = 1
    while k < s:
        fill_cs = jnp.full((b, k), -1, jnp.int32)
        fill_se = jnp.zeros((b, k), jnp.int32)
        cs = jnp.maximum(cs, jnp.concatenate([fill_cs, cs[:, : s - k]], axis=1))
        se = jnp.maximum(se, jnp.concatenate([fill_se, se[:, : s - k]], axis=1))
        k *= 2
    mask = (tok < 256) & (cs >= 0) & (se == 0)
    seq_pos = jnp.maximum(pos - cs - 1, 0)
    instr = lax.shift_right_logical(seq_pos * 52429, 18)
    byte_off = seq_pos - instr * 5
    addr = instr * 8 + byte_off
    addr_ref[...] = jnp.where(mask, addr, -1)


def _prep(token_ids):
    return pl.pallas_call(
        _prep_body,
        out_shape=jax.ShapeDtypeStruct(token_ids.shape, jnp.int32),
    )(token_ids)


def _patch_chunk(rows_v, addr_v, ci):
    ones = jnp.full((16,), 1.0, jnp.float32)
    for g in range(_CHUNK // 16):
        a = addr_v[pl.ds(ci * _CHUNK + g * 16, 16)]
        m = a >= 0
        lo = jnp.bitwise_and(a, 15)
        hi = jnp.bitwise_and(lax.shift_right_logical(a, 4), 15)
        top = jnp.bitwise_and(lax.shift_right_logical(a, 8), 15)
        row = lax.broadcasted_iota(jnp.int32, (16,), 0) + (g * 16)
        plsc.store_scatter(rows_v, [row, _ADDR_KEY + lo], ones, mask=m)
        plsc.store_scatter(rows_v, [row, _ADDR_KEY + 16 + hi], ones, mask=m)
        plsc.store_scatter(rows_v, [row, _ADDR_KEY + 32 + top], ones, mask=m)


def _sc_body(seq, w_hbm, tok_hbm, addr_hbm, out_hbm,
             w_sh, idx_v, addr_v, rows_list, gsem_list, osem_list):
    sid = lax.axis_index("s")
    wid = sid * _NC + lax.axis_index("c")
    bsz, seq_ = tok_hbm.shape
    n_per_w = (bsz * seq_) // _NW
    w_per_row = seq_ // n_per_w
    bi = wid // w_per_row
    s0 = (wid % w_per_row) * n_per_w
    nchunks = n_per_w // _CHUNK
    rows = rows_list
    gsem = gsem_list
    osem = osem_list

    pltpu.sync_copy(w_hbm.at[pl.ds(sid * 16, 16)], w_sh.at[pl.ds(sid * 16, 16)])

    @pl.when(sid == 0)
    def _tail():
        pltpu.sync_copy(w_hbm.at[pl.ds(_VOCAB - 16, 16)],
                        w_sh.at[pl.ds(_VOCAB - 16, 16)])

    pltpu.sync_copy(tok_hbm.at[bi, pl.ds(s0, n_per_w)], idx_v)
    pltpu.sync_copy(addr_hbm.at[bi, pl.ds(s0, n_per_w)], addr_v)
    plsc.subcore_barrier()

    def fire_gather(ci):
        b = ci % _NBUF
        base_t = ci * _CHUNK

        def group(g, carry):
            off = base_t + g * 16
            v = idx_v[pl.ds(off, 16)]
            tl = g * 16
            for k in range(16):
                pltpu.async_copy(w_sh.at[v[k]], rows[b].at[tl + k], gsem[b])
            return carry

        lax.fori_loop(0, _CHUNK // 16, group, 0)

    def wait_gather(ci):
        b = ci % _NBUF
        pltpu.make_async_copy(w_hbm.at[pl.ds(0, _CHUNK)], rows[b],
                              gsem[b]).wait()

    def out_slice(ci):
        return out_hbm.at[bi, pl.ds(s0 + ci * _CHUNK, _CHUNK)]

    def wait_out(ci):
        b = ci % _NBUF
        pltpu.make_async_copy(rows[b], out_slice(ci), osem[b]).wait()

    lead = _NBUF // 2
    for ci in range(lead):
        fire_gather(ci)
    for ci in range(nchunks):
        b = ci % _NBUF
        if ci >= lead:
            wait_out(ci - lead)
        if ci + lead < nchunks:
            fire_gather(ci + lead)
        wait_gather(ci)
        _patch_chunk(rows[b], addr_v, ci)
        pltpu.async_copy(rows[b], out_slice(ci), osem[b])
    for ci in range(nchunks - lead, nchunks):
        wait_out(ci)


def _sc_gather(w, tok, addr, bsz, seq):
    n_per_w = (bsz * seq) // _NW
    mesh = plsc.VectorSubcoreMesh(
        core_axis_name="c", subcore_axis_name="s",
        num_cores=_NC, num_subcores=_NS,
    )
    return pl.kernel(
        functools.partial(_sc_body, seq),
        out_type=jax.ShapeDtypeStruct((bsz, seq, _D), jnp.float32),
        mesh=mesh,
        compiler_params=pltpu.CompilerParams(
            use_tc_tiling_on_sc=True, needs_layout_passes=False),
        scratch_types=[
            pltpu.VMEM_SHARED((_VOCAB, _D), jnp.float32),
            pltpu.VMEM((n_per_w,), jnp.int32),
            pltpu.VMEM((n_per_w,), jnp.int32),
            [pltpu.VMEM((_CHUNK, _D), jnp.float32) for _ in range(_NBUF)],
            [pltpu.SemaphoreType.DMA for _ in range(_NBUF)],
            [pltpu.SemaphoreType.DMA for _ in range(_NBUF)],
        ],
    )(w, tok, addr)


def kernel(token_ids, W):
    bsz, seq = token_ids.shape
    tok = token_ids.astype(jnp.int32)
    addr = _prep(tok)
    return _sc_gather(W, tok, addr, bsz, seq)
